# baseline (speedup 1.0000x reference)
import functools

import jax
import jax.numpy as jnp
from jax import lax
from jax.experimental import pallas as pl
from jax.experimental.pallas import tpu as pltpu
from jax.experimental.pallas import tpu_sc as plsc

N_EDGES = 320000
N_NODES = 10000
D = 128

E = 512
EP = E + 8
CROWS = N_NODES + E + 16
SENT = CROWS - 1

NC = 2
NS = 16
NW = NC * NS
ROWS_PER_W = 384
CHUNK = 128
NCHUNK = ROWS_PER_W // CHUNK
BPAD = NW * ROWS_PER_W

_LOG2 = 0.6931471805599453


def _ssp(x):
    return jax.nn.softplus(x) - _LOG2


def _mlp_compact_body(starts_ref, rg_ref, x_ref, w1_ref, b1_ref, w2_ref,
                      b2_ref, out_ref):
    b = pl.program_id(0)

    @pl.when(b == 0)
    def _init():
        out_ref[...] = jnp.zeros_like(out_ref)

    x = x_ref[...]
    h = _ssp(jnp.dot(x, w1_ref[...], preferred_element_type=jnp.float32,
                     precision=lax.Precision.HIGHEST) + b1_ref[...])
    w = _ssp(jnp.dot(h, w2_ref[...], preferred_element_type=jnp.float32,
                     precision=lax.Precision.HIGHEST) + b2_ref[...])

    start = starts_ref[b]
    start_al = (start // 8) * 8
    rloc = rg_ref[0, 0, :] - start_al
    k_iota = lax.broadcasted_iota(jnp.int32, (EP, E), 0)
    p = (k_iota == rloc[None, :]).astype(jnp.float32)
    local = jnp.dot(p, w, preferred_element_type=jnp.float32,
                    precision=lax.Precision.HIGHEST)
    out_ref[pl.ds(start_al, EP), :] += local


def _mlp_compact(starts, rg3, dijk, w1, b1, w2, b2):
    nb = N_EDGES // E
    grid_spec = pltpu.PrefetchScalarGridSpec(
        num_scalar_prefetch=1,
        grid=(nb,),
        in_specs=[
            pl.BlockSpec((1, 1, E), lambda b, *_: (b, 0, 0)),
            pl.BlockSpec((E, D), lambda b, *_: (b, 0)),
            pl.BlockSpec((D, D), lambda b, *_: (0, 0)),
            pl.BlockSpec((1, D), lambda b, *_: (0, 0)),
            pl.BlockSpec((D, D), lambda b, *_: (0, 0)),
            pl.BlockSpec((1, D), lambda b, *_: (0, 0)),
        ],
        out_specs=pl.BlockSpec((CROWS, D), lambda b, *_: (0, 0)),
    )
    return pl.pallas_call(
        _mlp_compact_body,
        grid_spec=grid_spec,
        out_shape=jax.ShapeDtypeStruct((CROWS, D), jnp.float32),
    )(starts, rg3, dijk, w1, b1, w2, b2)


def _gather_body(table_hbm, idx_hbm, out_hbm, idx_v, rows_v, sem):
    wid = lax.axis_index("s") * NC + lax.axis_index("c")
    pltpu.sync_copy(idx_hbm.at[wid], idx_v)
    copies = [
        pltpu.async_copy(table_hbm.at[idx_v.at[j]],
                         rows_v.at[pl.ds(j * CHUNK, CHUNK)], sem)
        for j in range(NCHUNK)
    ]
    for c in copies:
        c.wait()
    pltpu.sync_copy(rows_v, out_hbm.at[pl.ds(wid * ROWS_PER_W, ROWS_PER_W)])


def _sc_gather(table, idx3):
    mesh = plsc.VectorSubcoreMesh(core_axis_name="c", subcore_axis_name="s")
    return pl.kernel(
        _gather_body,
        out_type=jax.ShapeDtypeStruct((BPAD, D), jnp.float32),
        mesh=mesh,
        scratch_types=[
            pltpu.VMEM((NCHUNK, CHUNK), jnp.int32),
            pltpu.VMEM((ROWS_PER_W, D), jnp.float32),
            pltpu.SemaphoreType.DMA,
        ],
    )(table, idx3)


def kernel(dijk, seg_j, W1, b1, W2, b2):
    seg = seg_j.astype(jnp.int32)
    flags = jnp.concatenate(
        [jnp.zeros((1,), jnp.int32),
         (seg[1:] != seg[:-1]).astype(jnp.int32)])
    rg = jnp.cumsum(flags, dtype=jnp.int32)
    starts = rg[::E]
    rg3 = rg.reshape(N_EDGES // E, 1, E)
    idx = jnp.full((BPAD,), SENT, jnp.int32).at[seg].set(rg)
    idx3 = idx.reshape(NW, NCHUNK, CHUNK)

    compact = _mlp_compact(starts, rg3, dijk,
                           W1, b1.reshape(1, D), W2, b2.reshape(1, D))
    outp = _sc_gather(compact, idx3)
    return outp[:N_NODES]


# speedup vs baseline: 1.3551x; 1.3551x over previous
import functools

import jax
import jax.numpy as jnp
from jax import lax
from jax.experimental import pallas as pl
from jax.experimental.pallas import tpu as pltpu
from jax.experimental.pallas import tpu_sc as plsc

N_EDGES = 320000
N_NODES = 10000
D = 128

E = 512
G = 128
GP = G + 8
CROWS = N_NODES + G + 16
SENT = CROWS - 1

NC = 2
NS = 16
NW = NC * NS
ROWS_PER_W = 384
CHUNK = 128
NCHUNK = ROWS_PER_W // CHUNK
BPAD = NW * ROWS_PER_W

_LOG2 = 0.6931471805599453
_BF16 = jnp.bfloat16


def _ssp(x):
    return jax.nn.softplus(x) - _LOG2


def _split(x):
    hi = x.astype(_BF16)
    lo = (x - hi.astype(jnp.float32)).astype(_BF16)
    return hi, lo


def _dot16(a, b):
    return jnp.dot(a, b, preferred_element_type=jnp.float32)


def _mm3(x, wh, wl):
    xh, xl = _split(x)
    return _dot16(xh, wh) + (_dot16(xh, wl) + _dot16(xl, wh))


def _mlp_compact_body(starts_ref, rg_ref, x_ref, w1h_ref, w1l_ref, b1_ref,
                      w2h_ref, w2l_ref, b2_ref, out_ref):
    b = pl.program_id(0)

    @pl.when(b == 0)
    def _init():
        out_ref[...] = jnp.zeros_like(out_ref)

    h = _ssp(_mm3(x_ref[...], w1h_ref[...], w1l_ref[...]) + b1_ref[...])
    w = _ssp(_mm3(h, w2h_ref[...], w2l_ref[...]) + b2_ref[...])
    wh, wl = _split(w)

    nsub = E // G
    for j in range(nsub):
        s = starts_ref[b * nsub + j]
        s_al = (s // 8) * 8
        rloc = rg_ref[0, 0, j * G:(j + 1) * G] - s_al
        k_iota = lax.broadcasted_iota(jnp.int32, (GP, G), 0)
        p = jnp.where(k_iota == rloc[None, :], 1.0, 0.0).astype(_BF16)
        local = _dot16(p, wh[j * G:(j + 1) * G]) + \
            _dot16(p, wl[j * G:(j + 1) * G])
        out_ref[pl.ds(s_al, GP), :] += local


def _mlp_compact(starts, rg3, dijk, w1h, w1l, b1, w2h, w2l, b2):
    nb = N_EDGES // E
    grid_spec = pltpu.PrefetchScalarGridSpec(
        num_scalar_prefetch=1,
        grid=(nb,),
        in_specs=[
            pl.BlockSpec((1, 1, E), lambda b, *_: (b, 0, 0)),
            pl.BlockSpec((E, D), lambda b, *_: (b, 0)),
            pl.BlockSpec((D, D), lambda b, *_: (0, 0)),
            pl.BlockSpec((D, D), lambda b, *_: (0, 0)),
            pl.BlockSpec((1, D), lambda b, *_: (0, 0)),
            pl.BlockSpec((D, D), lambda b, *_: (0, 0)),
            pl.BlockSpec((D, D), lambda b, *_: (0, 0)),
            pl.BlockSpec((1, D), lambda b, *_: (0, 0)),
        ],
        out_specs=pl.BlockSpec((CROWS, D), lambda b, *_: (0, 0)),
    )
    return pl.pallas_call(
        _mlp_compact_body,
        grid_spec=grid_spec,
        out_shape=jax.ShapeDtypeStruct((CROWS, D), jnp.float32),
    )(starts, rg3, dijk, w1h, w1l, b1, w2h, w2l, b2)


def _gather_body(table_hbm, idx_hbm, out_hbm, idx_v, rows_v, sem):
    wid = lax.axis_index("s") * NC + lax.axis_index("c")
    pltpu.sync_copy(idx_hbm.at[wid], idx_v)
    copies = [
        pltpu.async_copy(table_hbm.at[idx_v.at[j]],
                         rows_v.at[pl.ds(j * CHUNK, CHUNK)], sem)
        for j in range(NCHUNK)
    ]
    for c in copies:
        c.wait()
    pltpu.sync_copy(rows_v, out_hbm.at[pl.ds(wid * ROWS_PER_W, ROWS_PER_W)])


def _sc_gather(table, idx3):
    mesh = plsc.VectorSubcoreMesh(core_axis_name="c", subcore_axis_name="s")
    return pl.kernel(
        _gather_body,
        out_type=jax.ShapeDtypeStruct((BPAD, D), jnp.float32),
        mesh=mesh,
        scratch_types=[
            pltpu.VMEM((NCHUNK, CHUNK), jnp.int32),
            pltpu.VMEM((ROWS_PER_W, D), jnp.float32),
            pltpu.SemaphoreType.DMA,
        ],
    )(table, idx3)


def kernel(dijk, seg_j, W1, b1, W2, b2):
    seg = seg_j.astype(jnp.int32)
    flags = jnp.concatenate(
        [jnp.zeros((1,), jnp.int32),
         (seg[1:] != seg[:-1]).astype(jnp.int32)])
    rg = jnp.cumsum(flags, dtype=jnp.int32)
    starts = rg[::G]
    rg3 = rg.reshape(N_EDGES // E, 1, E)
    idx = jnp.full((BPAD,), SENT, jnp.int32).at[seg].set(rg)
    idx3 = idx.reshape(NW, NCHUNK, CHUNK)

    w1h, w1l = _split(W1)
    w2h, w2l = _split(W2)
    compact = _mlp_compact(starts, rg3, dijk, w1h, w1l, b1.reshape(1, D),
                           w2h, w2l, b2.reshape(1, D))
    outp = _sc_gather(compact, idx3)
    return outp[:N_NODES]


# speedup vs baseline: 3.2220x; 2.3777x over previous
import functools

import jax
import jax.numpy as jnp
from jax import lax
from jax.experimental import pallas as pl
from jax.experimental.pallas import tpu as pltpu
from jax.experimental.pallas import tpu_sc as plsc

N_EDGES = 320000
N_NODES = 10000
D = 128

E = 512
G = 128
GP = G + 8
CROWS = N_NODES + G + 16
SENT = CROWS - 1

NC = 2
NS = 16
NW = NC * NS
ROWS_PER_W = 384
CHUNK = 128
NCHUNK = ROWS_PER_W // CHUNK
BPAD = NW * ROWS_PER_W

_LOG2 = 0.6931471805599453
_BF16 = jnp.bfloat16


def _ssp(x):
    return jax.nn.softplus(x) - _LOG2


def _split(x):
    hi = x.astype(_BF16)
    lo = (x - hi.astype(jnp.float32)).astype(_BF16)
    return hi, lo


def _dot16(a, b):
    return jnp.dot(a, b, preferred_element_type=jnp.float32)


def _mm3(x, wh, wl):
    xh, xl = _split(x)
    return _dot16(xh, wh) + (_dot16(xh, wl) + _dot16(xl, wh))


def _mlp_compact_body(starts_ref, rg_ref, seg_ref, x_ref, w1h_ref, w1l_ref,
                      b1_ref, w2h_ref, w2l_ref, b2_ref, out_ref, uid_ref):
    b = pl.program_id(0)

    @pl.when(b == 0)
    def _init():
        out_ref[...] = jnp.zeros_like(out_ref)
        uid_ref[...] = jnp.zeros_like(uid_ref)

    h = _ssp(_mm3(x_ref[...], w1h_ref[...], w1l_ref[...]) + b1_ref[...])
    w = _ssp(_mm3(h, w2h_ref[...], w2l_ref[...]) + b2_ref[...])
    wh, wl = _split(w)

    nsub = E // G
    for j in range(nsub):
        sv = starts_ref[b * nsub + j]
        s = sv >> 1
        own0 = sv & 1
        s_al = (s // 8) * 8
        rloc = rg_ref[0, 0, j * G:(j + 1) * G] - s_al
        k_iota = lax.broadcasted_iota(jnp.int32, (GP, G), 0)
        pb = k_iota == rloc[None, :]
        p = jnp.where(pb, 1.0, 0.0).astype(_BF16)
        local = _dot16(p, wh[j * G:(j + 1) * G]) + \
            _dot16(p, wl[j * G:(j + 1) * G])
        out_ref[pl.ds(s_al, GP), :] += local
        segv = seg_ref[0, 0, j * G:(j + 1) * G]
        masked = jnp.where(pb, segv[None, :], -1)
        uidmax = jnp.max(masked, axis=1, keepdims=True)
        uidb = jnp.broadcast_to(uidmax, (GP, G))
        rloc0 = s - s_al
        own = (k_iota > rloc0) | ((k_iota == rloc0) & (own0 == 1))
        contrib = jnp.where(own, jnp.maximum(uidb + 1, 0), 0)
        uid_ref[pl.ds(s_al, GP), :] += contrib


def _mlp_compact(starts2, rg3, seg3, dijk, w1h, w1l, b1, w2h, w2l, b2):
    nb = N_EDGES // E
    grid_spec = pltpu.PrefetchScalarGridSpec(
        num_scalar_prefetch=1,
        grid=(nb,),
        in_specs=[
            pl.BlockSpec((1, 1, E), lambda b, *_: (b, 0, 0)),
            pl.BlockSpec((1, 1, E), lambda b, *_: (b, 0, 0)),
            pl.BlockSpec((E, D), lambda b, *_: (b, 0)),
            pl.BlockSpec((D, D), lambda b, *_: (0, 0)),
            pl.BlockSpec((D, D), lambda b, *_: (0, 0)),
            pl.BlockSpec((1, D), lambda b, *_: (0, 0)),
            pl.BlockSpec((D, D), lambda b, *_: (0, 0)),
            pl.BlockSpec((D, D), lambda b, *_: (0, 0)),
            pl.BlockSpec((1, D), lambda b, *_: (0, 0)),
        ],
        out_specs=[
            pl.BlockSpec((CROWS, D), lambda b, *_: (0, 0)),
            pl.BlockSpec((CROWS, D), lambda b, *_: (0, 0)),
        ],
    )
    return pl.pallas_call(
        _mlp_compact_body,
        grid_spec=grid_spec,
        out_shape=[
            jax.ShapeDtypeStruct((CROWS, D), jnp.float32),
            jax.ShapeDtypeStruct((CROWS, D), jnp.int32),
        ],
    )(starts2, rg3, seg3, dijk, w1h, w1l, b1, w2h, w2l, b2)


def _gather_body(table_hbm, idx_hbm, out_hbm, idx_v, rows_v, sem):
    wid = lax.axis_index("s") * NC + lax.axis_index("c")
    pltpu.sync_copy(idx_hbm.at[wid], idx_v)
    copies = [
        pltpu.async_copy(table_hbm.at[idx_v.at[j]],
                         rows_v.at[pl.ds(j * CHUNK, CHUNK)], sem)
        for j in range(NCHUNK)
    ]
    for c in copies:
        c.wait()
    pltpu.sync_copy(rows_v, out_hbm.at[pl.ds(wid * ROWS_PER_W, ROWS_PER_W)])


def _sc_gather(table, idx3):
    mesh = plsc.VectorSubcoreMesh(core_axis_name="c", subcore_axis_name="s")
    return pl.kernel(
        _gather_body,
        out_type=jax.ShapeDtypeStruct((BPAD, D), jnp.float32),
        mesh=mesh,
        scratch_types=[
            pltpu.VMEM((NCHUNK, CHUNK), jnp.int32),
            pltpu.VMEM((ROWS_PER_W, D), jnp.float32),
            pltpu.SemaphoreType.DMA,
        ],
    )(table, idx3)


def kernel(dijk, seg_j, W1, b1, W2, b2):
    seg = seg_j.astype(jnp.int32)
    flags = jnp.concatenate(
        [jnp.zeros((1,), jnp.int32),
         (seg[1:] != seg[:-1]).astype(jnp.int32)])
    rg = jnp.cumsum(flags, dtype=jnp.int32)
    starts = rg[::G]
    prev_last = jnp.concatenate([jnp.full((1,), -1, jnp.int32),
                                 rg[G - 1::G][:-1]])
    starts2 = starts * 2 + (prev_last != starts).astype(jnp.int32)
    rg3 = rg.reshape(N_EDGES // E, 1, E)
    seg3 = seg.reshape(N_EDGES // E, 1, E)

    w1h, w1l = _split(W1)
    w2h, w2l = _split(W2)
    compact, uid = _mlp_compact(starts2, rg3, seg3, dijk, w1h, w1l,
                                b1.reshape(1, D), w2h, w2l, b2.reshape(1, D))

    uid_col = uid[:, 0] - 1
    uid_tgt = jnp.where(uid_col >= 0, uid_col, BPAD - 1)
    ar = jnp.arange(BPAD, dtype=jnp.int32)
    idx = jnp.where(ar < N_NODES, SENT, ar % SENT)
    idx = idx.at[uid_tgt].set(jnp.arange(CROWS, dtype=jnp.int32))
    idx3 = idx.reshape(NW, NCHUNK, CHUNK)

    outp = _sc_gather(compact, idx3)
    return outp[:N_NODES]
